# baseline (speedup 1.0000x reference)
import functools

import jax
import jax.numpy as jnp
from jax.experimental import pallas as pl
from jax.experimental.pallas import tpu as pltpu

N = 10000
D_IN = 128
H1 = 64
H2 = 32
NUM_CLASSES = 10

BLK = 200
NB = N // BLK


def _gae_body(x_ref, adj_ref, w1_ref, w2_ref, w3_ref, wd_ref, bd_ref,
              pred_ref, mu_ref, lv_ref,
              xw1_ref, h1_ref, acc_ref):
    p = pl.program_id(0)
    i = pl.program_id(1)

    @pl.when(p == 0)
    def _phase0():
        @pl.when(i == 0)
        def _():
            xw1_ref[...] = jnp.dot(x_ref[...], w1_ref[...],
                                   preferred_element_type=jnp.float32)

        blk = jnp.dot(adj_ref[...], xw1_ref[...],
                      preferred_element_type=jnp.float32)
        h1_ref[pl.ds(i * BLK, BLK), :] = jnp.maximum(blk, 0.0)

    @pl.when(p == 1)
    def _phase1():
        z = jnp.dot(adj_ref[...], h1_ref[...],
                    preferred_element_type=jnp.float32)
        mu_ref[...] = jnp.dot(z, w2_ref[...],
                              preferred_element_type=jnp.float32)
        lv = jnp.dot(z, w3_ref[...], preferred_element_type=jnp.float32)
        lv_ref[...] = lv
        s = jnp.sum(lv, axis=0, keepdims=True)

        @pl.when(i == 0)
        def _():
            acc_ref[...] = s

        @pl.when(i > 0)
        def _():
            acc_ref[...] += s

        @pl.when(i == NB - 1)
        def _():
            h = acc_ref[...] * (1.0 / N)
            pred_ref[...] = (jnp.dot(h, wd_ref[...],
                                     preferred_element_type=jnp.float32)
                             + bd_ref[...])


def _adj_map(p, i):
    return (i + p * (NB - 1 - 2 * i), 0)


def _out_map(p, i):
    return (NB - 1 - i * p, 0)


@functools.partial(jax.jit, static_argnames=())
def kernel(x, adj, W1, W2, W3, Wd, bd):
    bd2 = bd.reshape(1, NUM_CLASSES)
    pred, mu, logvar = pl.pallas_call(
        _gae_body,
        grid=(2, NB),
        in_specs=[
            pl.BlockSpec((N, D_IN), lambda p, i: (0, 0)),
            pl.BlockSpec((BLK, N), _adj_map),
            pl.BlockSpec((D_IN, H1), lambda p, i: (0, 0)),
            pl.BlockSpec((H1, H2), lambda p, i: (0, 0)),
            pl.BlockSpec((H1, H2), lambda p, i: (0, 0)),
            pl.BlockSpec((H2, NUM_CLASSES), lambda p, i: (0, 0)),
            pl.BlockSpec((1, NUM_CLASSES), lambda p, i: (0, 0)),
        ],
        out_specs=[
            pl.BlockSpec((1, NUM_CLASSES), lambda p, i: (0, 0)),
            pl.BlockSpec((BLK, H2), _out_map),
            pl.BlockSpec((BLK, H2), _out_map),
        ],
        out_shape=[
            jax.ShapeDtypeStruct((1, NUM_CLASSES), jnp.float32),
            jax.ShapeDtypeStruct((N, H2), jnp.float32),
            jax.ShapeDtypeStruct((N, H2), jnp.float32),
        ],
        scratch_shapes=[
            pltpu.VMEM((N, H1), jnp.float32),
            pltpu.VMEM((N, H1), jnp.float32),
            pltpu.VMEM((1, H2), jnp.float32),
        ],
        compiler_params=pltpu.CompilerParams(
            dimension_semantics=("arbitrary", "arbitrary"),
        ),
    )(x, adj, W1, W2, W3, Wd, bd2)
    return (pred.reshape(NUM_CLASSES), mu, logvar)


# speedup vs baseline: 1.0310x; 1.0310x over previous
import functools

import jax
import jax.numpy as jnp
from jax.experimental import pallas as pl
from jax.experimental.pallas import tpu as pltpu

N = 10000
D_IN = 128
H1 = 64
H2 = 32
NUM_CLASSES = 10

BLK = 400
NB = N // BLK


def _gae_body(x_ref, adj_ref, w1_ref, w2_ref, w3_ref, wd_ref, bd_ref,
              pred_ref, mu_ref, lv_ref,
              xw1_ref, h1_ref, acc_ref):
    p = pl.program_id(0)
    i = pl.program_id(1)

    @pl.when(p == 0)
    def _phase0():
        @pl.when(i == 0)
        def _():
            xw1_ref[...] = jnp.dot(x_ref[...], w1_ref[...],
                                   preferred_element_type=jnp.float32
                                   ).astype(jnp.bfloat16)

        blk = jax.lax.dot_general(
            adj_ref[...], xw1_ref[...],
            (((1,), (0,)), ((), ())),
            preferred_element_type=jnp.float32)
        h1_ref[pl.ds(i * BLK, BLK), :] = jnp.maximum(blk, 0.0
                                                     ).astype(jnp.bfloat16)

    @pl.when(p == 1)
    def _phase1():
        z = jax.lax.dot_general(
            adj_ref[...], h1_ref[...],
            (((1,), (0,)), ((), ())),
            preferred_element_type=jnp.float32)
        mu_ref[...] = jnp.dot(z, w2_ref[...],
                              preferred_element_type=jnp.float32)
        lv = jnp.dot(z, w3_ref[...], preferred_element_type=jnp.float32)
        lv_ref[...] = lv
        s = jnp.sum(lv, axis=0, keepdims=True)

        @pl.when(i == 0)
        def _():
            acc_ref[...] = s

        @pl.when(i > 0)
        def _():
            acc_ref[...] += s

        @pl.when(i == NB - 1)
        def _():
            h = acc_ref[...] * (1.0 / N)
            pred_ref[...] = (jnp.dot(h, wd_ref[...],
                                     preferred_element_type=jnp.float32)
                             + bd_ref[...])


def _adj_map(p, i):
    return (i + p * (NB - 1 - 2 * i), 0)


def _out_map(p, i):
    return (NB - 1 - i * p, 0)


@functools.partial(jax.jit, static_argnames=())
def kernel(x, adj, W1, W2, W3, Wd, bd):
    bd2 = bd.reshape(1, NUM_CLASSES)
    pred, mu, logvar = pl.pallas_call(
        _gae_body,
        grid=(2, NB),
        in_specs=[
            pl.BlockSpec((N, D_IN), lambda p, i: (0, 0)),
            pl.BlockSpec((BLK, N), _adj_map),
            pl.BlockSpec((D_IN, H1), lambda p, i: (0, 0)),
            pl.BlockSpec((H1, H2), lambda p, i: (0, 0)),
            pl.BlockSpec((H1, H2), lambda p, i: (0, 0)),
            pl.BlockSpec((H2, NUM_CLASSES), lambda p, i: (0, 0)),
            pl.BlockSpec((1, NUM_CLASSES), lambda p, i: (0, 0)),
        ],
        out_specs=[
            pl.BlockSpec((1, NUM_CLASSES), lambda p, i: (0, 0)),
            pl.BlockSpec((BLK, H2), _out_map),
            pl.BlockSpec((BLK, H2), _out_map),
        ],
        out_shape=[
            jax.ShapeDtypeStruct((1, NUM_CLASSES), jnp.float32),
            jax.ShapeDtypeStruct((N, H2), jnp.float32),
            jax.ShapeDtypeStruct((N, H2), jnp.float32),
        ],
        scratch_shapes=[
            pltpu.VMEM((N, H1), jnp.bfloat16),
            pltpu.VMEM((N, H1), jnp.bfloat16),
            pltpu.VMEM((1, H2), jnp.float32),
        ],
        compiler_params=pltpu.CompilerParams(
            dimension_semantics=("arbitrary", "arbitrary"),
        ),
    )(x, adj, W1, W2, W3, Wd, bd2)
    return (pred.reshape(NUM_CLASSES), mu, logvar)


# speedup vs baseline: 1.0363x; 1.0052x over previous
import functools

import jax
import jax.numpy as jnp
from jax.experimental import pallas as pl
from jax.experimental.pallas import tpu as pltpu

N = 10000
D_IN = 128
H1 = 64
H2 = 32
NUM_CLASSES = 10

BLK = 400
NB = N // BLK


def _gae_body(x_ref, adj_ref, w1_ref, w2_ref, w3_ref, wd_ref, bd_ref,
              pred_ref, mu_ref, lv_ref,
              xw1_ref, h1_ref, acc_ref):
    p = pl.program_id(0)
    i = pl.program_id(1)

    @pl.when(p == 0)
    def _phase0():
        @pl.when(i == 0)
        def _():
            xw1_ref[...] = jnp.dot(x_ref[...], w1_ref[...],
                                   preferred_element_type=jnp.float32)

        blk = jnp.dot(adj_ref[...], xw1_ref[...],
                      preferred_element_type=jnp.float32)
        h1_ref[pl.ds(i * BLK, BLK), :] = jnp.maximum(blk, 0.0)

    @pl.when(p == 1)
    def _phase1():
        z = jnp.dot(adj_ref[...], h1_ref[...],
                    preferred_element_type=jnp.float32)
        mu_ref[...] = jnp.dot(z, w2_ref[...],
                              preferred_element_type=jnp.float32)
        lv = jnp.dot(z, w3_ref[...], preferred_element_type=jnp.float32)
        lv_ref[...] = lv
        s = jnp.sum(lv, axis=0, keepdims=True)

        @pl.when(i == 0)
        def _():
            acc_ref[...] = s

        @pl.when(i > 0)
        def _():
            acc_ref[...] += s

        @pl.when(i == NB - 1)
        def _():
            h = acc_ref[...] * (1.0 / N)
            pred_ref[...] = (jnp.dot(h, wd_ref[...],
                                     preferred_element_type=jnp.float32)
                             + bd_ref[...])


def _adj_map(p, i):
    return (i + p * (NB - 1 - 2 * i), 0)


def _out_map(p, i):
    return (NB - 1 - i * p, 0)


@functools.partial(jax.jit, static_argnames=())
def kernel(x, adj, W1, W2, W3, Wd, bd):
    bd2 = bd.reshape(1, NUM_CLASSES)
    pred, mu, logvar = pl.pallas_call(
        _gae_body,
        grid=(2, NB),
        in_specs=[
            pl.BlockSpec((N, D_IN), lambda p, i: (0, 0)),
            pl.BlockSpec((BLK, N), _adj_map),
            pl.BlockSpec((D_IN, H1), lambda p, i: (0, 0)),
            pl.BlockSpec((H1, H2), lambda p, i: (0, 0)),
            pl.BlockSpec((H1, H2), lambda p, i: (0, 0)),
            pl.BlockSpec((H2, NUM_CLASSES), lambda p, i: (0, 0)),
            pl.BlockSpec((1, NUM_CLASSES), lambda p, i: (0, 0)),
        ],
        out_specs=[
            pl.BlockSpec((1, NUM_CLASSES), lambda p, i: (0, 0)),
            pl.BlockSpec((BLK, H2), _out_map),
            pl.BlockSpec((BLK, H2), _out_map),
        ],
        out_shape=[
            jax.ShapeDtypeStruct((1, NUM_CLASSES), jnp.float32),
            jax.ShapeDtypeStruct((N, H2), jnp.float32),
            jax.ShapeDtypeStruct((N, H2), jnp.float32),
        ],
        scratch_shapes=[
            pltpu.VMEM((N, H1), jnp.float32),
            pltpu.VMEM((N, H1), jnp.float32),
            pltpu.VMEM((1, H2), jnp.float32),
        ],
        compiler_params=pltpu.CompilerParams(
            dimension_semantics=("arbitrary", "arbitrary"),
        ),
    )(x, adj, W1, W2, W3, Wd, bd2)
    return (pred.reshape(NUM_CLASSES), mu, logvar)
